# baseline (speedup 1.0000x reference)
import jax
import jax.numpy as jnp
from jax.experimental import pallas as pl
from jax.experimental.pallas import tpu as pltpu

_N_ROWS = 128
_N_COLS = 100000
_TILE = 4096
_GRID = (_N_COLS + _TILE - 1) // _TILE

_K0 = 0
_K1 = 42
_KS2 = _K0 ^ _K1 ^ 0x1BD11BDA
_TINY = float(jnp.finfo(jnp.float32).tiny)
_IMAX = 2**31 - 1


def _rotl(x, r):
    return (x << jnp.uint32(r)) | (x >> jnp.uint32(32 - r))


def _random_bits(x1):
    ks = (jnp.uint32(_K0), jnp.uint32(_K1), jnp.uint32(_KS2))
    rot_a = (13, 15, 26, 6)
    rot_b = (17, 29, 16, 24)
    x0 = jnp.zeros_like(x1) + ks[0]
    x1 = x1 + ks[1]
    for i in range(5):
        for r in rot_a if i % 2 == 0 else rot_b:
            x0 = x0 + x1
            x1 = _rotl(x1, r)
            x1 = x1 ^ x0
        x0 = x0 + ks[(i + 1) % 3]
        x1 = x1 + ks[(i + 2) % 3] + jnp.uint32(i + 1)
    return x0 ^ x1


def _gumbel_tile(col0):
    rows = jax.lax.broadcasted_iota(jnp.uint32, (_N_ROWS, _TILE), 0)
    cols = jax.lax.broadcasted_iota(jnp.uint32, (_N_ROWS, _TILE), 1)
    flat = rows * jnp.uint32(_N_COLS) + cols + col0.astype(jnp.uint32)
    bits = _random_bits(flat)
    fl = jax.lax.bitcast_convert_type(
        (bits >> jnp.uint32(9)) | jnp.uint32(0x3F800000), jnp.float32
    ) - jnp.float32(1.0)
    tiny = jnp.float32(_TINY)
    u = jnp.maximum(tiny, fl * (jnp.float32(1.0) - tiny) + tiny)
    return -jnp.log(-jnp.log(u))


def _body(logits_ref, out_ref, max_ref, idx_ref):
    j = pl.program_id(0)
    col0 = j * _TILE
    vals = logits_ref[...] + _gumbel_tile(col0)
    cids = jax.lax.broadcasted_iota(jnp.int32, (_N_ROWS, _TILE), 1) + col0
    vals = jnp.where(cids < _N_COLS, vals, -jnp.inf)

    m = jnp.max(vals, axis=1, keepdims=True)
    first = jnp.min(
        jnp.where(vals == m, cids, jnp.int32(_IMAX)), axis=1, keepdims=True
    )

    @pl.when(j == 0)
    def _():
        max_ref[...] = m
        idx_ref[...] = first

    @pl.when(j > 0)
    def _():
        better = m > max_ref[...]
        idx_ref[...] = jnp.where(better, first, idx_ref[...])
        max_ref[...] = jnp.where(better, m, max_ref[...])

    @pl.when(j == _GRID - 1)
    def _():
        out_ref[...] = idx_ref[...]


def kernel(logits):
    out = pl.pallas_call(
        _body,
        grid=(_GRID,),
        in_specs=[pl.BlockSpec((_N_ROWS, _TILE), lambda j: (0, j))],
        out_specs=pl.BlockSpec((_N_ROWS, 1), lambda j: (0, 0)),
        out_shape=jax.ShapeDtypeStruct((_N_ROWS, 1), jnp.int32),
        scratch_shapes=[
            pltpu.VMEM((_N_ROWS, 1), jnp.float32),
            pltpu.VMEM((_N_ROWS, 1), jnp.int32),
        ],
    )(logits)
    return out.astype(jnp.int64)


# speedup vs baseline: 1.0779x; 1.0779x over previous
import jax
import jax.numpy as jnp
from jax.experimental import pallas as pl
from jax.experimental.pallas import tpu as pltpu

_N_ROWS = 128
_N_COLS = 100000
_TILE = 1024
_GRID = (_N_COLS + _TILE - 1) // _TILE

_K0 = 0
_K1 = 42
_KS2 = _K0 ^ _K1 ^ 0x1BD11BDA
_TINY = float(jnp.finfo(jnp.float32).tiny)
_IMAX = 2**31 - 1


def _rotl(x, r):
    return (x << jnp.uint32(r)) | (x >> jnp.uint32(32 - r))


def _random_bits(x1):
    ks = (jnp.uint32(_K0), jnp.uint32(_K1), jnp.uint32(_KS2))
    rot_a = (13, 15, 26, 6)
    rot_b = (17, 29, 16, 24)
    x0 = jnp.zeros_like(x1) + ks[0]
    x1 = x1 + ks[1]
    for i in range(5):
        for r in rot_a if i % 2 == 0 else rot_b:
            x0 = x0 + x1
            x1 = _rotl(x1, r)
            x1 = x1 ^ x0
        x0 = x0 + ks[(i + 1) % 3]
        x1 = x1 + ks[(i + 2) % 3] + jnp.uint32(i + 1)
    return x0 ^ x1


def _gumbel_tile(col0):
    rows = jax.lax.broadcasted_iota(jnp.uint32, (_N_ROWS, _TILE), 0)
    cols = jax.lax.broadcasted_iota(jnp.uint32, (_N_ROWS, _TILE), 1)
    flat = rows * jnp.uint32(_N_COLS) + cols + col0.astype(jnp.uint32)
    bits = _random_bits(flat)
    fl = jax.lax.bitcast_convert_type(
        (bits >> jnp.uint32(9)) | jnp.uint32(0x3F800000), jnp.float32
    ) - jnp.float32(1.0)
    tiny = jnp.float32(_TINY)
    u = jnp.maximum(tiny, fl * (jnp.float32(1.0) - tiny) + tiny)
    return -jnp.log(-jnp.log(u))


def _body(logits_ref, out_ref, max_ref, idx_ref):
    j = pl.program_id(0)
    col0 = j * _TILE
    vals = logits_ref[...] + _gumbel_tile(col0)
    cids = jax.lax.broadcasted_iota(jnp.int32, (_N_ROWS, _TILE), 1) + col0
    vals = jnp.where(cids < _N_COLS, vals, -jnp.inf)

    m = jnp.max(vals, axis=1, keepdims=True)
    first = jnp.min(
        jnp.where(vals == m, cids, jnp.int32(_IMAX)), axis=1, keepdims=True
    )

    @pl.when(j == 0)
    def _():
        max_ref[...] = m
        idx_ref[...] = first

    @pl.when(j > 0)
    def _():
        better = m > max_ref[...]
        idx_ref[...] = jnp.where(better, first, idx_ref[...])
        max_ref[...] = jnp.where(better, m, max_ref[...])

    @pl.when(j == _GRID - 1)
    def _():
        out_ref[...] = idx_ref[...]


def kernel(logits):
    out = pl.pallas_call(
        _body,
        grid=(_GRID,),
        in_specs=[pl.BlockSpec((_N_ROWS, _TILE), lambda j: (0, j))],
        out_specs=pl.BlockSpec((_N_ROWS, 1), lambda j: (0, 0)),
        out_shape=jax.ShapeDtypeStruct((_N_ROWS, 1), jnp.int32),
        scratch_shapes=[
            pltpu.VMEM((_N_ROWS, 1), jnp.float32),
            pltpu.VMEM((_N_ROWS, 1), jnp.int32),
        ],
    )(logits)
    return out.astype(jnp.int64)


# speedup vs baseline: 1.1003x; 1.0208x over previous
import jax
import jax.numpy as jnp
from jax.experimental import pallas as pl
from jax.experimental.pallas import tpu as pltpu

_N_ROWS = 128
_N_COLS = 100000
_TILE = 2048
_GRID = (_N_COLS + _TILE - 1) // _TILE

_K0 = 0
_K1 = 42
_KS2 = _K0 ^ _K1 ^ 0x1BD11BDA
_TINY = float(jnp.finfo(jnp.float32).tiny)
_IMAX = 2**31 - 1


def _rotl(x, r):
    return (x << jnp.uint32(r)) | (x >> jnp.uint32(32 - r))


def _random_bits(x1):
    ks = (jnp.uint32(_K0), jnp.uint32(_K1), jnp.uint32(_KS2))
    rot_a = (13, 15, 26, 6)
    rot_b = (17, 29, 16, 24)
    x0 = jnp.zeros_like(x1) + ks[0]
    x1 = x1 + ks[1]
    for i in range(5):
        for r in rot_a if i % 2 == 0 else rot_b:
            x0 = x0 + x1
            x1 = _rotl(x1, r)
            x1 = x1 ^ x0
        x0 = x0 + ks[(i + 1) % 3]
        x1 = x1 + ks[(i + 2) % 3] + jnp.uint32(i + 1)
    return x0 ^ x1


def _gumbel_tile(col0):
    rows = jax.lax.broadcasted_iota(jnp.uint32, (_N_ROWS, _TILE), 0)
    cols = jax.lax.broadcasted_iota(jnp.uint32, (_N_ROWS, _TILE), 1)
    flat = rows * jnp.uint32(_N_COLS) + cols + col0.astype(jnp.uint32)
    bits = _random_bits(flat)
    fl = jax.lax.bitcast_convert_type(
        (bits >> jnp.uint32(9)) | jnp.uint32(0x3F800000), jnp.float32
    ) - jnp.float32(1.0)
    tiny = jnp.float32(_TINY)
    u = jnp.maximum(tiny, fl * (jnp.float32(1.0) - tiny) + tiny)
    return -jnp.log(-jnp.log(u))


def _body(logits_ref, out_ref, max_ref, idx_ref):
    j = pl.program_id(0)
    col0 = j * _TILE
    vals = logits_ref[...] + _gumbel_tile(col0)
    cids = jax.lax.broadcasted_iota(jnp.int32, (_N_ROWS, _TILE), 1) + col0
    vals = jnp.where(cids < _N_COLS, vals, -jnp.inf)

    m = jnp.max(vals, axis=1, keepdims=True)
    first = jnp.min(
        jnp.where(vals == m, cids, jnp.int32(_IMAX)), axis=1, keepdims=True
    )

    @pl.when(j == 0)
    def _():
        max_ref[...] = m
        idx_ref[...] = first

    @pl.when(j > 0)
    def _():
        better = m > max_ref[...]
        idx_ref[...] = jnp.where(better, first, idx_ref[...])
        max_ref[...] = jnp.where(better, m, max_ref[...])

    @pl.when(j == _GRID - 1)
    def _():
        out_ref[...] = idx_ref[...]


def kernel(logits):
    out = pl.pallas_call(
        _body,
        grid=(_GRID,),
        in_specs=[pl.BlockSpec((_N_ROWS, _TILE), lambda j: (0, j))],
        out_specs=pl.BlockSpec((_N_ROWS, 1), lambda j: (0, 0)),
        out_shape=jax.ShapeDtypeStruct((_N_ROWS, 1), jnp.int32),
        scratch_shapes=[
            pltpu.VMEM((_N_ROWS, 1), jnp.float32),
            pltpu.VMEM((_N_ROWS, 1), jnp.int32),
        ],
    )(logits)
    return out.astype(jnp.int64)


# speedup vs baseline: 1.1062x; 1.0054x over previous
import jax
import jax.numpy as jnp
from jax.experimental import pallas as pl
from jax.experimental.pallas import tpu as pltpu

_N_ROWS = 128
_N_COLS = 100000
_TILE = 2048
_GRID = (_N_COLS + _TILE - 1) // _TILE

_K0 = 0
_K1 = 42
_KS2 = _K0 ^ _K1 ^ 0x1BD11BDA
_TINY = float(jnp.finfo(jnp.float32).tiny)
_IMAX = 2**31 - 1


def _rotl(x, r):
    return (x << jnp.uint32(r)) | (x >> jnp.uint32(32 - r))


def _random_bits(x1):
    ks = (jnp.uint32(_K0), jnp.uint32(_K1), jnp.uint32(_KS2))
    rot_a = (13, 15, 26, 6)
    rot_b = (17, 29, 16, 24)
    x0 = jnp.zeros_like(x1) + ks[0]
    x1 = x1 + ks[1]
    for i in range(5):
        for r in rot_a if i % 2 == 0 else rot_b:
            x0 = x0 + x1
            x1 = _rotl(x1, r)
            x1 = x1 ^ x0
        x0 = x0 + ks[(i + 1) % 3]
        x1 = x1 + ks[(i + 2) % 3] + jnp.uint32(i + 1)
    return x0 ^ x1


def _gumbel_tile(col0):
    rows = jax.lax.broadcasted_iota(jnp.uint32, (_N_ROWS, _TILE), 0)
    cols = jax.lax.broadcasted_iota(jnp.uint32, (_N_ROWS, _TILE), 1)
    flat = rows * jnp.uint32(_N_COLS) + cols + col0.astype(jnp.uint32)
    bits = _random_bits(flat)
    fl = jax.lax.bitcast_convert_type(
        (bits >> jnp.uint32(9)) | jnp.uint32(0x3F800000), jnp.float32
    ) - jnp.float32(1.0)
    tiny = jnp.float32(_TINY)
    u = jnp.maximum(tiny, fl * (jnp.float32(1.0) - tiny) + tiny)
    return -jnp.log(-jnp.log(u))


_C1 = 896
_C2 = _TILE - _C1


def _gumbel_part(col0, width):
    rows = jax.lax.broadcasted_iota(jnp.uint32, (_N_ROWS, width), 0)
    cols = jax.lax.broadcasted_iota(jnp.uint32, (_N_ROWS, width), 1)
    flat = rows * jnp.uint32(_N_COLS) + cols + col0.astype(jnp.uint32)
    bits = _random_bits(flat)
    fl = jax.lax.bitcast_convert_type(
        (bits >> jnp.uint32(9)) | jnp.uint32(0x3F800000), jnp.float32
    ) - jnp.float32(1.0)
    tiny = jnp.float32(_TINY)
    u = jnp.maximum(tiny, fl * (jnp.float32(1.0) - tiny) + tiny)
    return -jnp.log(-jnp.log(u))


def _body(logits_ref, gc_ref, out_ref, max_ref, idx_ref):
    j = pl.program_id(0)
    col0 = j * _TILE
    vals_a = logits_ref[:, :_C1] + gc_ref[...]
    vals_b = logits_ref[:, _C1:] + _gumbel_part(col0 + _C1, _C2)
    vals = jnp.concatenate([vals_a, vals_b], axis=1)
    cids = jax.lax.broadcasted_iota(jnp.int32, (_N_ROWS, _TILE), 1) + col0
    vals = jnp.where(cids < _N_COLS, vals, -jnp.inf)

    m = jnp.max(vals, axis=1, keepdims=True)
    first = jnp.min(
        jnp.where(vals == m, cids, jnp.int32(_IMAX)), axis=1, keepdims=True
    )

    @pl.when(j == 0)
    def _():
        max_ref[...] = m
        idx_ref[...] = first

    @pl.when(j > 0)
    def _():
        better = m > max_ref[...]
        idx_ref[...] = jnp.where(better, first, idx_ref[...])
        max_ref[...] = jnp.where(better, m, max_ref[...])

    @pl.when(j == _GRID - 1)
    def _():
        out_ref[...] = idx_ref[...]


def _gen_body(g_ref):
    g_ref[...] = _gumbel_part(pl.program_id(0) * _TILE, _C1)


_GC_CACHE = None


def _gc_table():
    global _GC_CACHE
    if _GC_CACHE is None:
        _GC_CACHE = jax.jit(
            pl.pallas_call(
                _gen_body,
                grid=(_GRID,),
                out_specs=pl.BlockSpec((_N_ROWS, _C1), lambda j: (0, j)),
                out_shape=jax.ShapeDtypeStruct(
                    (_N_ROWS, _GRID * _C1), jnp.float32
                ),
            )
        )()
    return _GC_CACHE


def kernel(logits):
    gc = _gc_table()
    out = pl.pallas_call(
        _body,
        grid=(_GRID,),
        in_specs=[
            pl.BlockSpec((_N_ROWS, _TILE), lambda j: (0, j)),
            pl.BlockSpec((_N_ROWS, _C1), lambda j: (0, j)),
        ],
        out_specs=pl.BlockSpec((_N_ROWS, 1), lambda j: (0, 0)),
        out_shape=jax.ShapeDtypeStruct((_N_ROWS, 1), jnp.int32),
        scratch_shapes=[
            pltpu.VMEM((_N_ROWS, 1), jnp.float32),
            pltpu.VMEM((_N_ROWS, 1), jnp.int32),
        ],
    )(logits, gc)
    return out.astype(jnp.int64)
